# Initial kernel scaffold; baseline (speedup 1.0000x reference)
import jax
import jax.numpy as jnp
from jax.experimental import pallas as pl


def kernel(x, edge_index, W_rel, W_root, b):
    raise NotImplementedError("write your pallas kernel here")


# speedup vs baseline: 9.4300x; 9.4300x over previous
import jax
import jax.numpy as jnp
from jax import lax
from jax.experimental import pallas as pl
from jax.experimental.pallas import tpu as pltpu
from jax.experimental.pallas import tpu_sc as plsc

N_NODES = 10000
N_EDGES = 320000
D = 128

NC = 2
NS = 16
NW = NC * NS

EDGES_PER_TILE = N_EDGES // NW
CHUNK = 80
NCHUNK = EDGES_PER_TILE // CHUNK
PHASES = 5
PCHUNK = NCHUNK // PHASES
ROWS_MAIN = 624
ROWS_LAST = 640


def _sc_body(x_hbm, src_hbm, dst_hbm, out_hbm,
             acc, src_idx, dst_idx, rows0, rows1, sem0, sem1):
    c = lax.axis_index("c")
    s = lax.axis_index("s")
    w = c * NS + s

    z16 = jnp.zeros((16,), jnp.float32)

    def zb(i, carry):
        for j in range(D // 16):
            rows0[i, pl.ds(j * 16, 16)] = z16
        return carry

    lax.fori_loop(0, CHUNK, zb, 0)
    row0 = s * ROWS_MAIN
    for r in range(ROWS_LAST // CHUNK):
        pltpu.sync_copy(rows0, acc.at[pl.ds(row0 + r * CHUNK, CHUNK)])
    plsc.subcore_barrier()

    def phase(p, carry):
        pltpu.sync_copy(src_hbm.at[w, p], src_idx)
        pltpu.sync_copy(dst_hbm.at[w, p], dst_idx)
        pltpu.async_copy(x_hbm.at[src_idx.at[0]], rows0, sem0)

        def body(k, carry2):
            for b in range(2):
                kk = 2 * k + b
                rbuf, sem = (rows0, sem0) if b == 0 else (rows1, sem1)
                nbuf, nsem = (rows1, sem1) if b == 0 else (rows0, sem0)
                pltpu.make_async_copy(x_hbm.at[src_idx.at[kk]], rbuf, sem).wait()
                pltpu.async_copy(x_hbm.at[src_idx.at[kk + 1]], nbuf, nsem)
                pltpu.sync_copy(rbuf, acc.at[dst_idx.at[kk]], add=True)
            return carry2

        lax.fori_loop(0, (PCHUNK - 1) // 2, body, 0)
        last = PCHUNK - 1
        pltpu.make_async_copy(x_hbm.at[src_idx.at[last]], rows0, sem0).wait()
        pltpu.sync_copy(rows0, acc.at[dst_idx.at[last]], add=True)
        return carry

    lax.fori_loop(0, PHASES, phase, 0)

    plsc.subcore_barrier()

    @pl.when(s < NS - 1)
    def _():
        pltpu.sync_copy(acc.at[pl.ds(row0, ROWS_MAIN)],
                        out_hbm.at[pl.ds(c * N_NODES + row0, ROWS_MAIN)])

    @pl.when(s == NS - 1)
    def _():
        pltpu.sync_copy(acc.at[pl.ds(row0, ROWS_LAST)],
                        out_hbm.at[pl.ds(c * N_NODES + row0, ROWS_LAST)])


def _sc_scatter(x, src3, dst3):
    mesh = plsc.VectorSubcoreMesh(core_axis_name="c", subcore_axis_name="s")
    f = pl.kernel(
        _sc_body,
        out_type=jax.ShapeDtypeStruct((2 * N_NODES, D), jnp.float32),
        mesh=mesh,
        scratch_types=[
            pltpu.VMEM_SHARED((N_NODES, D), jnp.float32),
            pltpu.VMEM((PCHUNK, CHUNK), jnp.int32),
            pltpu.VMEM((PCHUNK, CHUNK), jnp.int32),
            pltpu.VMEM((CHUNK, D), jnp.float32),
            pltpu.VMEM((CHUNK, D), jnp.float32),
            pltpu.SemaphoreType.DMA,
            pltpu.SemaphoreType.DMA,
        ],
    )
    return f(x, src3, dst3)


def _tc_body(p0, p1, xr, wr, wo, bb, o):
    agg = p0[...] + p1[...]
    o[...] = (jnp.dot(agg, wr[...], preferred_element_type=jnp.float32)
              + jnp.dot(xr[...], wo[...], preferred_element_type=jnp.float32)
              + bb[...])


def _tc_combine(partial, x, wrT, woT, b2):
    mb = 1000
    grid = (N_NODES // mb,)
    return pl.pallas_call(
        _tc_body,
        grid=grid,
        in_specs=[
            pl.BlockSpec((mb, D), lambda i: (i, 0)),
            pl.BlockSpec((mb, D), lambda i: (i + N_NODES // mb, 0)),
            pl.BlockSpec((mb, D), lambda i: (i, 0)),
            pl.BlockSpec((D, D), lambda i: (0, 0)),
            pl.BlockSpec((D, D), lambda i: (0, 0)),
            pl.BlockSpec((1, D), lambda i: (0, 0)),
        ],
        out_specs=pl.BlockSpec((mb, D), lambda i: (i, 0)),
        out_shape=jax.ShapeDtypeStruct((N_NODES, D), jnp.float32),
    )(partial, partial, x, wrT, woT, b2)


def kernel(x, edge_index, W_rel, W_root, b):
    ei = edge_index.astype(jnp.int32)
    src3 = ei[0].reshape(NW, PHASES, PCHUNK, CHUNK)
    dst3 = ei[1].reshape(NW, PHASES, PCHUNK, CHUNK)
    partial = _sc_scatter(x, src3, dst3)
    return _tc_combine(partial, x, W_rel.T, W_root.T, b.reshape(1, D))


# speedup vs baseline: 13.0728x; 1.3863x over previous
import jax
import jax.numpy as jnp
from jax import lax
from jax.experimental import pallas as pl
from jax.experimental.pallas import tpu as pltpu
from jax.experimental.pallas import tpu_sc as plsc

N_NODES = 10000
N_EDGES = 320000
D = 128

NC = 2
NS = 16
NW = NC * NS

EDGES_PER_TILE = N_EDGES // NW
CHUNK = 80
NCHUNK = EDGES_PER_TILE // CHUNK
PHASES = 5
PCHUNK = NCHUNK // PHASES
ROWS_MAIN = 624
ROWS_LAST = 640


def _sc_body(x_hbm, src_hbm, dst_hbm, out_hbm,
             acc, src_idx, dst_idx, rows0, rows1, rows2, rows3,
             sem0, sem1, sem2, sem3):
    rows = (rows0, rows1, rows2, rows3)
    sems = (sem0, sem1, sem2, sem3)
    c = lax.axis_index("c")
    s = lax.axis_index("s")
    w = c * NS + s

    z16 = jnp.zeros((16,), jnp.float32)

    def zb(i, carry):
        for j in range(D // 16):
            rows0[i, pl.ds(j * 16, 16)] = z16
        return carry

    lax.fori_loop(0, CHUNK, zb, 0)
    row0 = s * ROWS_MAIN
    for r in range(ROWS_LAST // CHUNK):
        pltpu.sync_copy(rows0, acc.at[pl.ds(row0 + r * CHUNK, CHUNK)])
    plsc.subcore_barrier()

    def phase(p, carry):
        pltpu.sync_copy(src_hbm.at[w, p], src_idx)
        pltpu.sync_copy(dst_hbm.at[w, p], dst_idx)
        for j in range(3):
            pltpu.async_copy(x_hbm.at[src_idx.at[j]], rows[j], sems[j])

        def body(k, carry2):
            for b in range(4):
                kk = 4 * k + b
                pltpu.make_async_copy(
                    x_hbm.at[src_idx.at[kk]], rows[b], sems[b]).wait()
                nb = (b + 3) % 4
                pltpu.async_copy(
                    x_hbm.at[src_idx.at[kk + 3]], rows[nb], sems[nb])
                pltpu.sync_copy(rows[b], acc.at[dst_idx.at[kk]], add=True)
            return carry2

        lax.fori_loop(0, 5, body, 0)
        for kk in range(PCHUNK - 5, PCHUNK):
            b = kk % 4
            pltpu.make_async_copy(
                x_hbm.at[src_idx.at[kk]], rows[b], sems[b]).wait()
            if kk + 3 < PCHUNK:
                nb = (b + 3) % 4
                pltpu.async_copy(
                    x_hbm.at[src_idx.at[kk + 3]], rows[nb], sems[nb])
            pltpu.sync_copy(rows[b], acc.at[dst_idx.at[kk]], add=True)
        return carry

    lax.fori_loop(0, PHASES, phase, 0)

    plsc.subcore_barrier()

    @pl.when(s < NS - 1)
    def _():
        pltpu.sync_copy(acc.at[pl.ds(row0, ROWS_MAIN)],
                        out_hbm.at[pl.ds(c * N_NODES + row0, ROWS_MAIN)])

    @pl.when(s == NS - 1)
    def _():
        pltpu.sync_copy(acc.at[pl.ds(row0, ROWS_LAST)],
                        out_hbm.at[pl.ds(c * N_NODES + row0, ROWS_LAST)])


def _sc_scatter(x, src3, dst3):
    mesh = plsc.VectorSubcoreMesh(core_axis_name="c", subcore_axis_name="s")
    f = pl.kernel(
        _sc_body,
        out_type=jax.ShapeDtypeStruct((2 * N_NODES, D), jnp.float32),
        mesh=mesh,
        scratch_types=[
            pltpu.VMEM_SHARED((N_NODES, D), jnp.float32),
            pltpu.VMEM((PCHUNK, CHUNK), jnp.int32),
            pltpu.VMEM((PCHUNK, CHUNK), jnp.int32),
            pltpu.VMEM((CHUNK, D), jnp.float32),
            pltpu.VMEM((CHUNK, D), jnp.float32),
            pltpu.VMEM((CHUNK, D), jnp.float32),
            pltpu.VMEM((CHUNK, D), jnp.float32),
            pltpu.SemaphoreType.DMA,
            pltpu.SemaphoreType.DMA,
            pltpu.SemaphoreType.DMA,
            pltpu.SemaphoreType.DMA,
        ],
    )
    return f(x, src3, dst3)


def _tc_body(p0, p1, xr, wr, wo, bb, o):
    agg = p0[...] + p1[...]
    o[...] = (jnp.dot(agg, wr[...], preferred_element_type=jnp.float32)
              + jnp.dot(xr[...], wo[...], preferred_element_type=jnp.float32)
              + bb[...])


def _tc_combine(partial, x, wrT, woT, b2):
    mb = 1000
    grid = (N_NODES // mb,)
    return pl.pallas_call(
        _tc_body,
        grid=grid,
        in_specs=[
            pl.BlockSpec((mb, D), lambda i: (i, 0)),
            pl.BlockSpec((mb, D), lambda i: (i + N_NODES // mb, 0)),
            pl.BlockSpec((mb, D), lambda i: (i, 0)),
            pl.BlockSpec((D, D), lambda i: (0, 0)),
            pl.BlockSpec((D, D), lambda i: (0, 0)),
            pl.BlockSpec((1, D), lambda i: (0, 0)),
        ],
        out_specs=pl.BlockSpec((mb, D), lambda i: (i, 0)),
        out_shape=jax.ShapeDtypeStruct((N_NODES, D), jnp.float32),
    )(partial, partial, x, wrT, woT, b2)


def kernel(x, edge_index, W_rel, W_root, b):
    ei = edge_index.astype(jnp.int32)
    src3 = ei[0].reshape(NW, PHASES, PCHUNK, CHUNK)
    dst3 = ei[1].reshape(NW, PHASES, PCHUNK, CHUNK)
    partial = _sc_scatter(x, src3, dst3)
    return _tc_combine(partial, x, W_rel.T, W_root.T, b.reshape(1, D))
